# baseline (speedup 1.0000x reference)
import functools

import jax
import jax.numpy as jnp
import numpy as np
from jax.experimental import pallas as pl
from jax.experimental.pallas import tpu as pltpu

B, T, P, D = 8, 128, 32, 32
K = 64
H = 4
NL = 2
L = T + P
E = L * D
DH = K // H
BP = 2

_fdot = jnp.dot
_INV_SQRT_K = 1.0 / np.sqrt(K)


def _selectors():
    j = np.arange(K)
    d = np.arange(D)
    rows = (d[:, None, None] * K + j[None, :, None])
    cols = np.arange(D * H)
    sel = ((rows // K) == (cols // H)) & ((j[None, :, None] // DH) == (cols % H))
    sel = sel.reshape(D * K, D * H).astype(np.float32)
    return sel, sel.T.copy()


def _b16(x):
    return x.astype(jnp.bfloat16).astype(jnp.float32)


def _ln(g, b, v):
    mu = jnp.mean(v, axis=-1, keepdims=True)
    c = v - mu
    var = jnp.mean(c * c, axis=-1, keepdims=True)
    return c / jnp.sqrt(var + 1e-5) * g + b


def _mab_tail(p, q2d, o2d):
    o = _ln(p["ln0_g"], p["ln0_b"], q2d + o2d)
    o = o + jax.nn.relu(_fdot(o, p["fc_o"]["w"]) + p["fc_o"]["b"])
    return _ln(p["ln1_g"], p["ln1_b"], o)


def _attend(q4, ku4, vu4, kn, vn, bk, bv, sel, selT, softmax_axis):
    k4 = ku4 + kn + bk
    v4 = vu4 + vn + bv
    prod = (_b16(k4) * _b16(q4)).reshape(BP * L, D * K)
    ph = _b16(prod)
    s = (_fdot(ph, sel) + _fdot(prod - ph, sel)) * _INV_SQRT_K
    s3 = s.reshape(BP, L, D, H)
    s3 = s3 - jnp.max(s3, axis=softmax_axis, keepdims=True)
    e3 = jnp.exp(s3)
    a = e3 / jnp.sum(e3, axis=softmax_axis, keepdims=True)
    a16 = _b16(a.reshape(BP * L, D * H))
    a4 = _fdot(a16, selT).reshape(BP, L, D, K)
    return jnp.sum(a4 * _b16(v4), axis=softmax_axis)


def _encoder_body(treedef, n_params, marks_ref, uf_ref, tgtm_ref,
                  sel_ref, selT_ref, *refs):
    param_refs, out_ref = refs[:n_params], refs[n_params]
    prm = jax.tree.unflatten(treedef, [r[...] for r in param_refs])
    sel, selT = sel_ref[...], selT_ref[...]

    marks = marks_ref[...]
    uf = uf_ref[...]
    tgtm = tgtm_ref[...]

    T2d = jax.nn.relu(marks * prm["time_init"]["w"]
                      + prm["time_init"]["b"]).reshape(BP * L, K)
    C2d = jnp.tile(jax.nn.relu(_b16(prm["chan_init"]["w"])
                               + prm["chan_init"]["b"]), (BP, 1))
    we = prm["edge_init"]["w"]
    U2d = jax.nn.relu(uf[:, :, :, None] * we[0].reshape(1, 1, 1, K)
                      + tgtm[:, :, :, None] * we[1].reshape(1, 1, 1, K)
                      + prm["edge_init"]["b"].reshape(1, 1, 1, K)).reshape(BP * E, K)

    for lp in prm["layers"]:
        cp, tp, wn = lp["ct_attn"], lp["tc_attn"], lp["edge_nn"]["w"]
        wkv = jnp.concatenate([cp["fc_k"]["w"][K:], cp["fc_v"]["w"][K:],
                               tp["fc_k"]["w"][K:], tp["fc_v"]["w"][K:],
                               wn[:K]], axis=1)
        kv = _fdot(U2d, wkv)
        kv4 = kv.reshape(BP, L, D, 5 * K)

        qc = _fdot(C2d, cp["fc_q"]["w"]) + cp["fc_q"]["b"]
        kt = _fdot(T2d, cp["fc_k"]["w"][:K]).reshape(BP, L, 1, K)
        vt = _fdot(T2d, cp["fc_v"]["w"][:K]).reshape(BP, L, 1, K)
        o = _attend(qc.reshape(BP, 1, D, K), kv4[..., :K], kv4[..., K:2 * K],
                    kt, vt, cp["fc_k"]["b"], cp["fc_v"]["b"], sel, selT, 1)
        C2d = _mab_tail(cp, qc, o.reshape(BP * D, K))

        qt = _fdot(T2d, tp["fc_q"]["w"]) + tp["fc_q"]["b"]
        kc = _fdot(C2d, tp["fc_k"]["w"][:K]).reshape(BP, 1, D, K)
        vc = _fdot(C2d, tp["fc_v"]["w"][:K]).reshape(BP, 1, D, K)
        o = _attend(qt.reshape(BP, L, 1, K), kv4[..., 2 * K:3 * K],
                    kv4[..., 3 * K:4 * K], kc, vc,
                    tp["fc_k"]["b"], tp["fc_v"]["b"], sel, selT, 2)
        T2d = _mab_tail(tp, qt, o.reshape(BP * L, K))

        bn = lp["edge_nn"]["b"]
        cat4 = (kv4[..., 4 * K:]
                + _fdot(T2d, wn[K:2 * K]).reshape(BP, L, 1, K)
                + (_fdot(C2d, wn[2 * K:]) + bn).reshape(BP, 1, D, K))
        U2d = jax.nn.relu(cat4).reshape(BP * E, K) + U2d

    wo, bo = prm["output"]["w"], prm["output"]["b"]
    u_term = jnp.sum(_b16(U2d.reshape(BP, L, D, K))
                     * _b16(wo[:K, 0]).reshape(1, 1, 1, K), axis=-1)
    t_term = jnp.sum(_b16(T2d) * _b16(wo[K:2 * K, 0])[None, :],
                     axis=-1).reshape(BP, L, 1)
    c_term = jnp.sum(_b16(C2d) * _b16(wo[2 * K:, 0])[None, :],
                     axis=-1).reshape(BP, 1, D)
    out_ref[...] = u_term + t_term + c_term + bo[0, 0]


def kernel(x, x_mark, x_mask, y, y_mark, y_mask, params):
    marks = jnp.concatenate([x_mark[:, :, 0], y_mark[:, :, 0]], axis=1)[:, :, None]
    xv = x * x_mask
    uf = jnp.concatenate([xv, jnp.zeros((B, P, D), jnp.float32)], axis=1)
    tgtm = jnp.concatenate([jnp.zeros((B, T, D), jnp.float32), y_mask], axis=1)

    params2 = jax.tree.map(lambda a: a.reshape(1, -1) if a.ndim == 1 else a, params)
    leaves, treedef = jax.tree.flatten(params2)
    sel, selT = (jnp.asarray(m) for m in _selectors())

    body = functools.partial(_encoder_body, treedef, len(leaves))
    bs_param = [pl.BlockSpec(l.shape, lambda b: (0,) * l.ndim) for l in leaves]
    out = pl.pallas_call(
        body,
        grid=(B // BP,),
        in_specs=[
            pl.BlockSpec((BP, L, 1), lambda b: (b, 0, 0)),
            pl.BlockSpec((BP, L, D), lambda b: (b, 0, 0)),
            pl.BlockSpec((BP, L, D), lambda b: (b, 0, 0)),
            pl.BlockSpec((D * K, D * H), lambda b: (0, 0)),
            pl.BlockSpec((D * H, D * K), lambda b: (0, 0)),
        ] + bs_param,
        out_specs=pl.BlockSpec((BP, L, D), lambda b: (b, 0, 0)),
        out_shape=jax.ShapeDtypeStruct((B, L, D), jnp.float32),
        compiler_params=pltpu.CompilerParams(
            dimension_semantics=("arbitrary",),
            vmem_limit_bytes=110 * 1024 * 1024,
        ),
    )(marks, uf, tgtm, sel, selT, *leaves)

    pred = out.reshape(B, E)
    tgt_u = jnp.concatenate([jnp.zeros((B, T, D), jnp.float32), y], axis=1).reshape(B, E)
    tgt_m = tgtm.reshape(B, E)
    return pred, tgt_u, tgt_m


# speedup vs baseline: 1.0447x; 1.0447x over previous
import functools

import jax
import jax.numpy as jnp
import numpy as np
from jax.experimental import pallas as pl
from jax.experimental.pallas import tpu as pltpu

B, T, P, D = 8, 128, 32, 32
K = 64
H = 4
NL = 2
L = T + P
E = L * D
DH = K // H
BP = 2

_fdot = jnp.dot
_INV_SQRT_K = 1.0 / np.sqrt(K)


def _selectors():
    j = np.arange(K)
    d = np.arange(D)
    rows = (d[:, None, None] * K + j[None, :, None])
    cols = np.arange(D * H)
    sel = ((rows // K) == (cols // H)) & ((j[None, :, None] // DH) == (cols % H))
    sel = sel.reshape(D * K, D * H).astype(np.float32)
    return sel, sel.T.copy()


def _b16(x):
    return x.astype(jnp.bfloat16).astype(jnp.float32)


def _ln(g, b, v):
    mu = jnp.mean(v, axis=-1, keepdims=True)
    c = v - mu
    var = jnp.mean(c * c, axis=-1, keepdims=True)
    return c / jnp.sqrt(var + 1e-5) * g + b


def _mab_tail(p, q2d, o2d):
    o = _ln(p["ln0_g"], p["ln0_b"], q2d + o2d)
    o = o + jax.nn.relu(_fdot(o, p["fc_o"]["w"]) + p["fc_o"]["b"])
    return _ln(p["ln1_g"], p["ln1_b"], o)


def _attend(q4, ku4, vu4, kn, vn, bk, bv, sel, selT, softmax_axis):
    k4 = ku4 + kn + bk
    v4 = vu4 + vn + bv
    prod = (k4 * q4).reshape(BP * L, D * K)
    s = _fdot(prod, sel) * _INV_SQRT_K
    s3 = s.reshape(BP, L, D, H)
    s3 = s3 - jnp.max(s3, axis=softmax_axis, keepdims=True)
    e3 = jnp.exp(s3)
    a = e3 / jnp.sum(e3, axis=softmax_axis, keepdims=True)
    a16 = _b16(a.reshape(BP * L, D * H))
    a4 = _fdot(a16, selT).reshape(BP, L, D, K)
    return jnp.sum(a4 * _b16(v4), axis=softmax_axis)


def _encoder_body(treedef, n_params, marks_ref, uf_ref, tgtm_ref,
                  sel_ref, selT_ref, *refs):
    param_refs, out_ref = refs[:n_params], refs[n_params]
    prm = jax.tree.unflatten(treedef, [r[...] for r in param_refs])
    sel, selT = sel_ref[...], selT_ref[...]

    marks = marks_ref[...]
    uf = uf_ref[...]
    tgtm = tgtm_ref[...]

    T2d = jax.nn.relu(marks * prm["time_init"]["w"]
                      + prm["time_init"]["b"]).reshape(BP * L, K)
    C2d = jnp.tile(jax.nn.relu(_b16(prm["chan_init"]["w"])
                               + prm["chan_init"]["b"]), (BP, 1))
    we = prm["edge_init"]["w"]
    U2d = jax.nn.relu(uf[:, :, :, None] * we[0].reshape(1, 1, 1, K)
                      + tgtm[:, :, :, None] * we[1].reshape(1, 1, 1, K)
                      + prm["edge_init"]["b"].reshape(1, 1, 1, K)).reshape(BP * E, K)

    for lp in prm["layers"]:
        cp, tp, wn = lp["ct_attn"], lp["tc_attn"], lp["edge_nn"]["w"]
        wkv = jnp.concatenate([cp["fc_k"]["w"][K:], cp["fc_v"]["w"][K:],
                               tp["fc_k"]["w"][K:], tp["fc_v"]["w"][K:],
                               wn[:K]], axis=1)
        kv = _fdot(U2d, wkv)
        kv4 = kv.reshape(BP, L, D, 5 * K)

        qc = _fdot(C2d, cp["fc_q"]["w"]) + cp["fc_q"]["b"]
        kt = _fdot(T2d, cp["fc_k"]["w"][:K]).reshape(BP, L, 1, K)
        vt = _fdot(T2d, cp["fc_v"]["w"][:K]).reshape(BP, L, 1, K)
        o = _attend(qc.reshape(BP, 1, D, K), kv4[..., :K], kv4[..., K:2 * K],
                    kt, vt, cp["fc_k"]["b"], cp["fc_v"]["b"], sel, selT, 1)
        C2d = _mab_tail(cp, qc, o.reshape(BP * D, K))

        qt = _fdot(T2d, tp["fc_q"]["w"]) + tp["fc_q"]["b"]
        kc = _fdot(C2d, tp["fc_k"]["w"][:K]).reshape(BP, 1, D, K)
        vc = _fdot(C2d, tp["fc_v"]["w"][:K]).reshape(BP, 1, D, K)
        o = _attend(qt.reshape(BP, L, 1, K), kv4[..., 2 * K:3 * K],
                    kv4[..., 3 * K:4 * K], kc, vc,
                    tp["fc_k"]["b"], tp["fc_v"]["b"], sel, selT, 2)
        T2d = _mab_tail(tp, qt, o.reshape(BP * L, K))

        bn = lp["edge_nn"]["b"]
        cat4 = (kv4[..., 4 * K:]
                + _fdot(T2d, wn[K:2 * K]).reshape(BP, L, 1, K)
                + (_fdot(C2d, wn[2 * K:]) + bn).reshape(BP, 1, D, K))
        U2d = jax.nn.relu(cat4).reshape(BP * E, K) + U2d

    wo, bo = prm["output"]["w"], prm["output"]["b"]
    u_term = jnp.sum(_b16(U2d.reshape(BP, L, D, K))
                     * _b16(wo[:K, 0]).reshape(1, 1, 1, K), axis=-1)
    t_term = jnp.sum(_b16(T2d) * _b16(wo[K:2 * K, 0])[None, :],
                     axis=-1).reshape(BP, L, 1)
    c_term = jnp.sum(_b16(C2d) * _b16(wo[2 * K:, 0])[None, :],
                     axis=-1).reshape(BP, 1, D)
    out_ref[...] = u_term + t_term + c_term + bo[0, 0]


def kernel(x, x_mark, x_mask, y, y_mark, y_mask, params):
    marks = jnp.concatenate([x_mark[:, :, 0], y_mark[:, :, 0]], axis=1)[:, :, None]
    xv = x * x_mask
    uf = jnp.concatenate([xv, jnp.zeros((B, P, D), jnp.float32)], axis=1)
    tgtm = jnp.concatenate([jnp.zeros((B, T, D), jnp.float32), y_mask], axis=1)

    params2 = jax.tree.map(lambda a: a.reshape(1, -1) if a.ndim == 1 else a, params)
    leaves, treedef = jax.tree.flatten(params2)
    sel, selT = (jnp.asarray(m) for m in _selectors())

    body = functools.partial(_encoder_body, treedef, len(leaves))
    bs_param = [pl.BlockSpec(l.shape, lambda b: (0,) * l.ndim) for l in leaves]
    out = pl.pallas_call(
        body,
        grid=(B // BP,),
        in_specs=[
            pl.BlockSpec((BP, L, 1), lambda b: (b, 0, 0)),
            pl.BlockSpec((BP, L, D), lambda b: (b, 0, 0)),
            pl.BlockSpec((BP, L, D), lambda b: (b, 0, 0)),
            pl.BlockSpec((D * K, D * H), lambda b: (0, 0)),
            pl.BlockSpec((D * H, D * K), lambda b: (0, 0)),
        ] + bs_param,
        out_specs=pl.BlockSpec((BP, L, D), lambda b: (b, 0, 0)),
        out_shape=jax.ShapeDtypeStruct((B, L, D), jnp.float32),
        compiler_params=pltpu.CompilerParams(
            dimension_semantics=("arbitrary",),
            vmem_limit_bytes=110 * 1024 * 1024,
        ),
    )(marks, uf, tgtm, sel, selT, *leaves)

    pred = out.reshape(B, E)
    tgt_u = jnp.concatenate([jnp.zeros((B, T, D), jnp.float32), y], axis=1).reshape(B, E)
    tgt_m = tgtm.reshape(B, E)
    return pred, tgt_u, tgt_m


# speedup vs baseline: 1.0506x; 1.0056x over previous
import functools

import jax
import jax.numpy as jnp
import numpy as np
from jax.experimental import pallas as pl
from jax.experimental.pallas import tpu as pltpu

B, T, P, D = 8, 128, 32, 32
K = 64
H = 4
NL = 2
L = T + P
E = L * D
DH = K // H
BP = 2

_fdot = jnp.dot
_INV_SQRT_K = 1.0 / np.sqrt(K)


def _selectors():
    j = np.arange(K)
    d = np.arange(D)
    rows = (d[:, None, None] * K + j[None, :, None])
    cols = np.arange(D * H)
    sel = ((rows // K) == (cols // H)) & ((j[None, :, None] // DH) == (cols % H))
    sel = sel.reshape(D * K, D * H).astype(np.float32)
    return sel, sel.T.copy()


def _b16(x):
    return x.astype(jnp.bfloat16).astype(jnp.float32)


def _ln(g, b, v):
    mu = jnp.mean(v, axis=-1, keepdims=True)
    c = v - mu
    var = jnp.mean(c * c, axis=-1, keepdims=True)
    return c / jnp.sqrt(var + 1e-5) * g + b


def _mab_tail(p, q2d, o2d):
    o = _ln(p["ln0_g"], p["ln0_b"], q2d + o2d)
    o = o + jax.nn.relu(_fdot(o, p["fc_o"]["w"]) + p["fc_o"]["b"])
    return _ln(p["ln1_g"], p["ln1_b"], o)


def _attend(q4, ku4, vu4, kn, vn, bk, bv, sel, selT, softmax_axis):
    k4 = ku4 + kn + bk
    v4 = vu4 + vn + bv
    prod = (k4 * q4).reshape(BP * L, D * K)
    s = _fdot(prod, sel) * _INV_SQRT_K
    s3 = s.reshape(BP, L, D, H)
    s3 = s3 - jnp.max(s3, axis=softmax_axis, keepdims=True)
    e3 = jnp.exp(s3)
    a = e3 / jnp.sum(e3, axis=softmax_axis, keepdims=True)
    a16 = _b16(a.reshape(BP * L, D * H))
    a4 = _fdot(a16, selT).reshape(BP, L, D, K)
    return jnp.sum(a4 * _b16(v4), axis=softmax_axis)


def _encoder_body(treedef, n_params, marks_ref, uf_ref, tgtm_ref,
                  sel_ref, selT_ref, *refs):
    param_refs, out_ref = refs[:n_params], refs[n_params]
    prm = jax.tree.unflatten(treedef, [r[...] for r in param_refs])
    sel, selT = sel_ref[...], selT_ref[...]

    marks = marks_ref[...]
    uf = uf_ref[...]
    tgtm = tgtm_ref[...]

    T2d = jax.nn.relu(marks * prm["time_init"]["w"]
                      + prm["time_init"]["b"]).reshape(BP * L, K)
    C2d = jnp.tile(jax.nn.relu(_b16(prm["chan_init"]["w"])
                               + prm["chan_init"]["b"]), (BP, 1))
    we = prm["edge_init"]["w"]
    U2d = jax.nn.relu(uf[:, :, :, None] * we[0].reshape(1, 1, 1, K)
                      + tgtm[:, :, :, None] * we[1].reshape(1, 1, 1, K)
                      + prm["edge_init"]["b"].reshape(1, 1, 1, K)).reshape(BP * E, K)

    for lp in prm["layers"]:
        cp, tp, wn = lp["ct_attn"], lp["tc_attn"], lp["edge_nn"]["w"]
        ku_c = _fdot(U2d, cp["fc_k"]["w"][K:]).reshape(BP, L, D, K)
        vu_c = _fdot(U2d, cp["fc_v"]["w"][K:]).reshape(BP, L, D, K)
        ku_t = _fdot(U2d, tp["fc_k"]["w"][K:]).reshape(BP, L, D, K)
        vu_t = _fdot(U2d, tp["fc_v"]["w"][K:]).reshape(BP, L, D, K)
        eu = _fdot(U2d, wn[:K]).reshape(BP, L, D, K)

        qc = _fdot(C2d, cp["fc_q"]["w"]) + cp["fc_q"]["b"]
        kt = _fdot(T2d, cp["fc_k"]["w"][:K]).reshape(BP, L, 1, K)
        vt = _fdot(T2d, cp["fc_v"]["w"][:K]).reshape(BP, L, 1, K)
        o = _attend(qc.reshape(BP, 1, D, K), ku_c, vu_c,
                    kt, vt, cp["fc_k"]["b"], cp["fc_v"]["b"], sel, selT, 1)
        C2d = _mab_tail(cp, qc, o.reshape(BP * D, K))

        qt = _fdot(T2d, tp["fc_q"]["w"]) + tp["fc_q"]["b"]
        kc = _fdot(C2d, tp["fc_k"]["w"][:K]).reshape(BP, 1, D, K)
        vc = _fdot(C2d, tp["fc_v"]["w"][:K]).reshape(BP, 1, D, K)
        o = _attend(qt.reshape(BP, L, 1, K), ku_t, vu_t, kc, vc,
                    tp["fc_k"]["b"], tp["fc_v"]["b"], sel, selT, 2)
        T2d = _mab_tail(tp, qt, o.reshape(BP * L, K))

        bn = lp["edge_nn"]["b"]
        cat4 = (eu
                + _fdot(T2d, wn[K:2 * K]).reshape(BP, L, 1, K)
                + (_fdot(C2d, wn[2 * K:]) + bn).reshape(BP, 1, D, K))
        U2d = jax.nn.relu(cat4).reshape(BP * E, K) + U2d

    wo, bo = prm["output"]["w"], prm["output"]["b"]
    u_term = jnp.sum(_b16(U2d.reshape(BP, L, D, K))
                     * _b16(wo[:K, 0]).reshape(1, 1, 1, K), axis=-1)
    t_term = jnp.sum(_b16(T2d) * _b16(wo[K:2 * K, 0])[None, :],
                     axis=-1).reshape(BP, L, 1)
    c_term = jnp.sum(_b16(C2d) * _b16(wo[2 * K:, 0])[None, :],
                     axis=-1).reshape(BP, 1, D)
    out_ref[...] = u_term + t_term + c_term + bo[0, 0]


def kernel(x, x_mark, x_mask, y, y_mark, y_mask, params):
    marks = jnp.concatenate([x_mark[:, :, 0], y_mark[:, :, 0]], axis=1)[:, :, None]
    xv = x * x_mask
    uf = jnp.concatenate([xv, jnp.zeros((B, P, D), jnp.float32)], axis=1)
    tgtm = jnp.concatenate([jnp.zeros((B, T, D), jnp.float32), y_mask], axis=1)

    params2 = jax.tree.map(lambda a: a.reshape(1, -1) if a.ndim == 1 else a, params)
    leaves, treedef = jax.tree.flatten(params2)
    sel, selT = (jnp.asarray(m) for m in _selectors())

    body = functools.partial(_encoder_body, treedef, len(leaves))
    bs_param = [pl.BlockSpec(l.shape, lambda b: (0,) * l.ndim) for l in leaves]
    out = pl.pallas_call(
        body,
        grid=(B // BP,),
        in_specs=[
            pl.BlockSpec((BP, L, 1), lambda b: (b, 0, 0)),
            pl.BlockSpec((BP, L, D), lambda b: (b, 0, 0)),
            pl.BlockSpec((BP, L, D), lambda b: (b, 0, 0)),
            pl.BlockSpec((D * K, D * H), lambda b: (0, 0)),
            pl.BlockSpec((D * H, D * K), lambda b: (0, 0)),
        ] + bs_param,
        out_specs=pl.BlockSpec((BP, L, D), lambda b: (b, 0, 0)),
        out_shape=jax.ShapeDtypeStruct((B, L, D), jnp.float32),
        compiler_params=pltpu.CompilerParams(
            dimension_semantics=("arbitrary",),
            vmem_limit_bytes=110 * 1024 * 1024,
        ),
    )(marks, uf, tgtm, sel, selT, *leaves)

    pred = out.reshape(B, E)
    tgt_u = jnp.concatenate([jnp.zeros((B, T, D), jnp.float32), y], axis=1).reshape(B, E)
    tgt_m = tgtm.reshape(B, E)
    return pred, tgt_u, tgt_m
